# Initial kernel scaffold; baseline (speedup 1.0000x reference)
import jax
import jax.numpy as jnp
from jax.experimental import pallas as pl


def kernel(x, table, bias):
    raise NotImplementedError("write your pallas kernel here")


# speedup vs baseline: 16.0180x; 16.0180x over previous
import functools

import jax
import jax.numpy as jnp
from jax import lax
from jax.experimental import pallas as pl
from jax.experimental.pallas import tpu as pltpu
from jax.experimental.pallas import tpu_sc as plsc

B = 16384
L = 200
D = 32
HALF = 16

NC = 2
NS = 16
NW = NC * NS

ROWS_PER_W = B // NW
CH = 8
NCHUNK = ROWS_PER_W // CH
UNROLL = 8


def _body(x_hbm, table_hbm, bias_hbm, out_hbm,
          idx0, idx1, rows0, rows1, outb, biasb, sem0, sem1):
    wid = lax.axis_index("s") * NC + lax.axis_index("c")
    base_row = wid * ROWS_PER_W

    pltpu.sync_copy(bias_hbm, biasb)
    b_lo = biasb[pl.ds(0, HALF)]
    b_hi = biasb[pl.ds(HALF, HALF)]

    idx_bufs = (idx0, idx1)
    rows_bufs = (rows0, rows1)
    sems = (sem0, sem1)

    def fire(c, b):
        pltpu.sync_copy(x_hbm.at[pl.ds((base_row + c * CH) * L, CH * L)],
                        idx_bufs[b])
        pltpu.async_copy(table_hbm.at[idx_bufs[b]], rows_bufs[b], sems[b])

    for b in range(2):
        fire(b, b)

    zero = jnp.zeros((HALF,), jnp.float32)

    def do_chunk(cur, b):
        pltpu.make_async_copy(table_hbm.at[idx_bufs[b]], rows_bufs[b],
                              sems[b]).wait()
        rows = rows_bufs[b]
        for i in range(CH):
            def red(t, carry):
                s0, s1, s2, s3 = carry
                r = i * L + t * UNROLL
                for u in range(UNROLL):
                    lo = rows[r + u, pl.ds(0, HALF)]
                    hi = rows[r + u, pl.ds(HALF, HALF)]
                    if u % 2 == 0:
                        s0 = s0 + lo
                        s1 = s1 + hi
                    else:
                        s2 = s2 + lo
                        s3 = s3 + hi
                return s0, s1, s2, s3

            s0, s1, s2, s3 = lax.fori_loop(0, L // UNROLL, red,
                                           (zero, zero, zero, zero))
            outb[pl.ds(i * D, HALF)] = s0 + s2 + b_lo
            outb[pl.ds(i * D + HALF, HALF)] = s1 + s3 + b_hi
        pltpu.sync_copy(outb,
                        out_hbm.at[pl.ds((base_row + cur * CH) * D, CH * D)])

        @pl.when(cur + 2 < NCHUNK)
        def _():
            fire(cur + 2, b)

    def outer(g2, carry):
        do_chunk(g2 * 2, 0)
        do_chunk(g2 * 2 + 1, 1)
        return carry

    lax.fori_loop(0, NCHUNK // 2, outer, 0)


@jax.jit
def kernel(x, table, bias):
    mesh = plsc.VectorSubcoreMesh(core_axis_name="c", subcore_axis_name="s")
    run = functools.partial(
        pl.kernel,
        mesh=mesh,
        compiler_params=pltpu.CompilerParams(use_tc_tiling_on_sc=False),
        out_type=jax.ShapeDtypeStruct((B * D,), jnp.float32),
        scratch_types=[
            pltpu.VMEM((CH * L,), jnp.int32),
            pltpu.VMEM((CH * L,), jnp.int32),
            pltpu.VMEM((CH * L, D), jnp.float32),
            pltpu.VMEM((CH * L, D), jnp.float32),
            pltpu.VMEM((CH * D,), jnp.float32),
            pltpu.VMEM((D,), jnp.float32),
            pltpu.SemaphoreType.DMA,
            pltpu.SemaphoreType.DMA,
        ],
    )(_body)
    out = run(x.reshape(B * L), table, bias)
    out2d = out.reshape(B, D)
    return (out2d[:, :HALF], out2d[:, HALF:])
